# Initial kernel scaffold; baseline (speedup 1.0000x reference)
import jax
import jax.numpy as jnp
from jax.experimental import pallas as pl


def kernel(x, edge_index, W1, b1, W2, b2, W3, b3, W4, b4):
    raise NotImplementedError("write your pallas kernel here")


# speedup vs baseline: 5.2556x; 5.2556x over previous
import functools

import jax
import jax.numpy as jnp
from jax import lax
from jax.experimental import pallas as pl
from jax.experimental.pallas import tpu as pltpu
from jax.experimental.pallas import tpu_sc as plsc

_N = 10000
_E = 160000
_NP = 10240
_EP = 163840
_DIN = 256
_DH = 512
_DOUT = 256

_mesh = plsc.VectorSubcoreMesh(core_axis_name="c", subcore_axis_name="s")


def _deg_call(dstp):
    epw = _EP // 32
    nb = epw // 128
    rpt = _NP // 16

    @functools.partial(
        pl.kernel,
        mesh=_mesh,
        out_type=jax.ShapeDtypeStruct((2 * _NP, 128), jnp.float32),
        scratch_types=[
            pltpu.VMEM((nb, 128), jnp.int32),
            pltpu.VMEM((128, 128), jnp.float32),
            pltpu.VMEM((16, 128), jnp.float32),
            pltpu.VMEM_SHARED((_NP, 128), jnp.float32),
        ],
    )
    def _deg(dst_h, out_h, dst_v, ones_v, zrows, acc):
        cid = lax.axis_index("c")
        sid = lax.axis_index("s")
        wid = cid * 16 + sid
        pltpu.sync_copy(dst_h.at[pl.ds(wid * nb, nb)], dst_v)
        one16 = jnp.ones((16,), jnp.float32)
        zero16 = jnp.zeros((16,), jnp.float32)
        for r in range(16):
            for j in range(8):
                zrows[r, pl.ds(j * 16, 16)] = zero16
        for r in range(128):
            for j in range(8):
                ones_v[r, pl.ds(j * 16, 16)] = one16
        rbase = sid * rpt
        def zb(i, _):
            pltpu.sync_copy(zrows, acc.at[pl.ds(rbase + i * 16, 16)])
            return 0
        lax.fori_loop(0, rpt // 16, zb, 0)
        plsc.subcore_barrier()
        def bb(b, _):
            pltpu.sync_copy(ones_v, acc.at[dst_v.at[b]], add=True)
            return 0
        lax.fori_loop(0, nb, bb, 0)
        plsc.subcore_barrier()
        pltpu.sync_copy(acc.at[pl.ds(rbase, rpt)],
                        out_h.at[pl.ds(cid * _NP + rbase, rpt)])

    return _deg(dstp).reshape(2, _NP, 128)


def _scatter_call(g_chunks, srcp, dstp):
    ept = _EP // 16
    nb = ept // 128
    rpt = _NP // 16

    @functools.partial(
        pl.kernel,
        mesh=_mesh,
        out_type=jax.ShapeDtypeStruct((_NP, _DH), jnp.float32),
        scratch_types=[
            pltpu.VMEM((nb, 128), jnp.int32),
            pltpu.VMEM((nb, 128), jnp.int32),
            pltpu.VMEM((128, 128), jnp.float32),
            pltpu.VMEM((16, 128), jnp.float32),
            pltpu.VMEM_SHARED((_NP, 128), jnp.float32),
            pltpu.SemaphoreType.DMA,
        ],
    )
    def _scat(g0h, g1h, g2h, g3h, src_h, dst_h, out_h,
              src_v, dst_v, rows_v, zrows, acc, sem):
        cid = lax.axis_index("c")
        sid = lax.axis_index("s")
        pltpu.sync_copy(src_h.at[pl.ds(sid * nb, nb)], src_v)
        pltpu.sync_copy(dst_h.at[pl.ds(sid * nb, nb)], dst_v)
        zero16 = jnp.zeros((16,), jnp.float32)
        for r in range(16):
            for j in range(8):
                zrows[r, pl.ds(j * 16, 16)] = zero16
        rbase = sid * rpt

        def chunk(gh, col):
            def zb(i, _):
                pltpu.sync_copy(zrows, acc.at[pl.ds(rbase + i * 16, 16)])
                return 0
            lax.fori_loop(0, rpt // 16, zb, 0)
            plsc.subcore_barrier()
            def bb(b, _):
                pltpu.async_copy(gh.at[src_v.at[b]], rows_v, sem).wait()
                pltpu.sync_copy(rows_v, acc.at[dst_v.at[b]], add=True)
                return 0
            lax.fori_loop(0, nb, bb, 0)
            plsc.subcore_barrier()
            pltpu.sync_copy(acc.at[pl.ds(rbase, rpt)],
                            out_h.at[pl.ds(rbase, rpt), pl.ds(col * 128, 128)])
            plsc.subcore_barrier()

        for k in range(4):
            gh = (g0h, g1h, g2h, g3h)[k]
            @pl.when(cid == (k // 2))
            def _(gh=gh, k=k):
                chunk(gh, k)

    return _scat(*g_chunks, srcp, dstp)


def _edge_gather_call(a, b, srcp, dstp):
    ept = _EP // 32
    nb = ept // 128

    @functools.partial(
        pl.kernel,
        mesh=_mesh,
        out_type=[jax.ShapeDtypeStruct((_EP, _DH), jnp.float32),
                  jax.ShapeDtypeStruct((_EP, _DH), jnp.float32)],
        scratch_types=[
            pltpu.VMEM((nb, 128), jnp.int32),
            pltpu.VMEM((nb, 128), jnp.int32),
            pltpu.VMEM((128, _DH), jnp.float32),
            pltpu.SemaphoreType.DMA,
        ],
    )
    def _eg(a_h, b_h, src_h, dst_h, ea_h, eb_h,
            src_v, dst_v, rows_v, sem):
        cid = lax.axis_index("c")
        sid = lax.axis_index("s")
        wid = cid * 16 + sid
        ebase = wid * ept
        pltpu.sync_copy(src_h.at[pl.ds(wid * nb, nb)], src_v)
        pltpu.sync_copy(dst_h.at[pl.ds(wid * nb, nb)], dst_v)
        def bb(i, _):
            pltpu.async_copy(a_h.at[src_v.at[i]], rows_v, sem).wait()
            pltpu.sync_copy(rows_v, ea_h.at[pl.ds(ebase + i * 128, 128)])
            pltpu.async_copy(b_h.at[dst_v.at[i]], rows_v, sem).wait()
            pltpu.sync_copy(rows_v, eb_h.at[pl.ds(ebase + i * 128, 128)])
            return 0
        lax.fori_loop(0, nb, bb, 0)

    return _eg(a, b, srcp, dstp)


def _dinv_from(degp_ref):
    deg = degp_ref[0, :, 0] + degp_ref[1, :, 0] + 1.0
    return lax.rsqrt(deg)[:, None]


def _mm1_body(degp_ref, x_ref, w_ref, o0, o1, o2, o3):
    dinv = _dinv_from(degp_ref)
    h = jnp.dot(x_ref[...], w_ref[...], preferred_element_type=jnp.float32)
    g = h * dinv
    o0[...] = g[:, 0:128]
    o1[...] = g[:, 128:256]
    o2[...] = g[:, 256:384]
    o3[...] = g[:, 384:512]


def _mm1_call(degp, xp, w1):
    grid = (_NP // 256,)
    cspec = pl.BlockSpec((256, 128), lambda i: (i, 0))
    return pl.pallas_call(
        _mm1_body,
        grid=grid,
        in_specs=[
            pl.BlockSpec((2, 256, 128), lambda i: (0, i, 0)),
            pl.BlockSpec((256, _DIN), lambda i: (i, 0)),
            pl.BlockSpec((_DIN, _DH), lambda i: (0, 0)),
        ],
        out_specs=[cspec, cspec, cspec, cspec],
        out_shape=[jax.ShapeDtypeStruct((_NP, 128), jnp.float32)] * 4,
    )(degp, xp, w1)


def _mm2_body(degp_ref, s_ref, g0, g1, g2, g3, b_ref, w_ref, o0, o1, o2, o3):
    dinv = _dinv_from(degp_ref)
    g = jnp.concatenate([g0[...], g1[...], g2[...], g3[...]], axis=1)
    h1 = jnp.maximum(dinv * (s_ref[...] + g) + b_ref[0:1, :], 0.0)
    h2 = jnp.dot(h1, w_ref[...], preferred_element_type=jnp.float32) * dinv
    o0[...] = h2[:, 0:128]
    o1[...] = h2[:, 128:256]
    o2[...] = h2[:, 256:384]
    o3[...] = h2[:, 384:512]


def _mm2_call(degp, s1, g_chunks, b1, w2):
    grid = (_NP // 256,)
    cspec = pl.BlockSpec((256, 128), lambda i: (i, 0))
    return pl.pallas_call(
        _mm2_body,
        grid=grid,
        in_specs=[
            pl.BlockSpec((2, 256, 128), lambda i: (0, i, 0)),
            pl.BlockSpec((256, _DH), lambda i: (i, 0)),
            cspec, cspec, cspec, cspec,
            pl.BlockSpec((8, _DH), lambda i: (0, 0)),
            pl.BlockSpec((_DH, _DH), lambda i: (0, 0)),
        ],
        out_specs=[cspec, cspec, cspec, cspec],
        out_shape=[jax.ShapeDtypeStruct((_NP, 128), jnp.float32)] * 4,
    )(degp, s1, *g_chunks, b1, w2)


def _mm3_body(degp_ref, s_ref, g0, g1, g2, g3, b2_ref, w3a_ref, w3b_ref,
              b3_ref, a_ref, bout_ref):
    dinv = _dinv_from(degp_ref)
    g = jnp.concatenate([g0[...], g1[...], g2[...], g3[...]], axis=1)
    h2 = jnp.maximum(dinv * (s_ref[...] + g) + b2_ref[0:1, :], 0.0)
    a_ref[...] = (jnp.dot(h2, w3a_ref[...], preferred_element_type=jnp.float32)
                  + b3_ref[0:1, :])
    bout_ref[...] = jnp.dot(h2, w3b_ref[...], preferred_element_type=jnp.float32)


def _mm3_call(degp, s2, g_chunks, b2, w3a, w3b, b3):
    grid = (_NP // 256,)
    cspec = pl.BlockSpec((256, 128), lambda i: (i, 0))
    fspec = pl.BlockSpec((256, _DH), lambda i: (i, 0))
    return pl.pallas_call(
        _mm3_body,
        grid=grid,
        in_specs=[
            pl.BlockSpec((2, 256, 128), lambda i: (0, i, 0)),
            fspec,
            cspec, cspec, cspec, cspec,
            pl.BlockSpec((8, _DH), lambda i: (0, 0)),
            pl.BlockSpec((_DH, _DH), lambda i: (0, 0)),
            pl.BlockSpec((_DH, _DH), lambda i: (0, 0)),
            pl.BlockSpec((8, _DH), lambda i: (0, 0)),
        ],
        out_specs=[fspec, fspec],
        out_shape=[jax.ShapeDtypeStruct((_NP, _DH), jnp.float32)] * 2,
    )(degp, s2, *g_chunks, b2, w3a, w3b, b3)


def _mm4_body(ea_ref, eb_ref, w_ref, b_ref, o_ref):
    eh = jnp.maximum(ea_ref[...] + eb_ref[...], 0.0)
    o_ref[...] = (jnp.dot(eh, w_ref[...], preferred_element_type=jnp.float32)
                  + b_ref[0:1, :])


def _mm4_call(ea, eb, w4, b4):
    grid = (_EP // 512,)
    espec = pl.BlockSpec((512, _DH), lambda i: (i, 0))
    return pl.pallas_call(
        _mm4_body,
        grid=grid,
        in_specs=[
            espec,
            espec,
            pl.BlockSpec((_DH, _DOUT), lambda i: (0, 0)),
            pl.BlockSpec((8, _DOUT), lambda i: (0, 0)),
        ],
        out_specs=pl.BlockSpec((512, _DOUT), lambda i: (i, 0)),
        out_shape=jax.ShapeDtypeStruct((_EP, _DOUT), jnp.float32),
    )(ea, eb, w4, b4)


def kernel(x, edge_index, W1, b1, W2, b2, W3, b3, W4, b4):
    xp = jnp.zeros((_NP, _DIN), jnp.float32).at[:_N].set(x)
    pad = _N + (jnp.arange(_EP - _E, dtype=jnp.int32) % (_NP - _N))
    srcp = jnp.concatenate([edge_index[0].astype(jnp.int32), pad])
    srcp = srcp.reshape(_EP // 128, 128)
    dstp = jnp.concatenate([edge_index[1].astype(jnp.int32), pad])
    dstp = dstp.reshape(_EP // 128, 128)
    b1r = jnp.broadcast_to(b1[None, :], (8, _DH))
    b2r = jnp.broadcast_to(b2[None, :], (8, _DH))
    b3r = jnp.broadcast_to(b3[None, :], (8, _DH))
    b4r = jnp.broadcast_to(b4[None, :], (8, _DOUT))

    degp = _deg_call(dstp)
    g1c = _mm1_call(degp, xp, W1)
    s1 = _scatter_call(g1c, srcp, dstp)
    g2c = _mm2_call(degp, s1, g1c, b1r, W2)
    s2 = _scatter_call(g2c, srcp, dstp)
    a, b = _mm3_call(degp, s2, g2c, b2r, W3[:_DH], W3[_DH:], b3r)
    ea, eb = _edge_gather_call(a, b, srcp, dstp)
    out = _mm4_call(ea, eb, W4, b4r)
    return out[:_E]
